# Initial kernel scaffold; baseline (speedup 1.0000x reference)
import jax
import jax.numpy as jnp
from jax.experimental import pallas as pl


def kernel(x, edge_index, W1, b1, W2, b2):
    raise NotImplementedError("write your pallas kernel here")


# speedup vs baseline: 16.0275x; 16.0275x over previous
import functools

import jax
import jax.numpy as jnp
from jax import lax
from jax.experimental import pallas as pl
from jax.experimental.pallas import tpu as pltpu
from jax.experimental.pallas import tpu_sc as plsc

N = 10000
D = 128
M = 64
E = 320000

NCORES = 2
NSUB = 16
NW = NCORES * NSUB
CHUNK = 128
CPT = 80
EPAD = NW * CPT * CHUNK
NPAD = 10240
SHARE = NPAD // NSUB
RB = 1000


def _mesh():
    return plsc.VectorSubcoreMesh(core_axis_name="c", subcore_axis_name="s")


def _sc_deg_body(dst_hbm, out_hbm, idx_v, ones_v, zline_v, acc_sh):
    cid = lax.axis_index("c")
    sid = lax.axis_index("s")
    wid = sid * NCORES + cid

    for i in range(CHUNK // 16):
        ones_v[pl.ds(i * 16, 16)] = jnp.ones((16,), jnp.float32)

    def _z(i, carry):
        zline_v[pl.ds(i * 16, 16)] = jnp.zeros((16,), jnp.float32)
        return carry

    lax.fori_loop(0, SHARE // 16, _z, 0)
    pltpu.sync_copy(zline_v, acc_sh.at[pl.ds(sid * SHARE, SHARE)])
    pltpu.sync_copy(dst_hbm.at[pl.ds(wid * CPT, CPT), :], idx_v)
    plsc.subcore_barrier()

    def _body(j, carry):
        pltpu.sync_copy(ones_v, acc_sh.at[idx_v.at[j]], add=True)
        return carry

    lax.fori_loop(0, CPT, _body, 0)
    plsc.subcore_barrier()
    pltpu.sync_copy(acc_sh.at[pl.ds(sid * SHARE, SHARE)],
                    out_hbm.at[cid, pl.ds(sid * SHARE, SHARE)])


def _sc_deg(dst2d):
    return pl.kernel(
        _sc_deg_body,
        out_type=jax.ShapeDtypeStruct((NCORES, NPAD), jnp.float32),
        mesh=_mesh(),
        scratch_types=[
            pltpu.VMEM((CPT, CHUNK), jnp.int32),
            pltpu.VMEM((CHUNK,), jnp.float32),
            pltpu.VMEM((SHARE,), jnp.float32),
            pltpu.VMEM_SHARED((NPAD,), jnp.float32),
        ],
    )(dst2d)


def _sc_agg_body(g_hbm, src_hbm, dst_hbm, out_hbm,
                 sidx_v, didx_v, rows_v, zrow_v, acc_sh, sem0, sem1):
    cid = lax.axis_index("c")
    sid = lax.axis_index("s")
    wid = sid * NCORES + cid
    sems = (sem0, sem1)

    def _z(i, carry):
        for k in range(M // 16):
            zrow_v[i, pl.ds(k * 16, 16)] = jnp.zeros((16,), jnp.float32)
        return carry

    lax.fori_loop(0, CHUNK, _z, 0)
    for k in range(SHARE // CHUNK):
        pltpu.sync_copy(zrow_v,
                        acc_sh.at[pl.ds(sid * SHARE + k * CHUNK, CHUNK), :])
    pltpu.sync_copy(src_hbm.at[pl.ds(wid * CPT, CPT), :], sidx_v)
    pltpu.sync_copy(dst_hbm.at[pl.ds(wid * CPT, CPT), :], didx_v)
    plsc.subcore_barrier()

    def _start(j, b):
        pltpu.make_async_copy(g_hbm.at[sidx_v.at[j]], rows_v.at[b],
                              sems[b]).start()

    def _finish(j, b):
        pltpu.make_async_copy(g_hbm.at[sidx_v.at[j]], rows_v.at[b],
                              sems[b]).wait()
        pltpu.sync_copy(rows_v.at[b], acc_sh.at[didx_v.at[j]], add=True)

    _start(0, 0)
    _start(1, 1)

    def _body(jo, carry):
        for b in range(2):
            j = jo * 2 + b
            _finish(j, b)
            _start(j + 2, b)
        return carry

    lax.fori_loop(0, CPT // 2 - 1, _body, 0)
    _finish(CPT - 2, 0)
    _finish(CPT - 1, 1)
    plsc.subcore_barrier()
    pltpu.sync_copy(acc_sh.at[pl.ds(sid * SHARE, SHARE), :],
                    out_hbm.at[cid, pl.ds(sid * SHARE, SHARE), :])


def _sc_agg(g, src2d, dst2d):
    return pl.kernel(
        _sc_agg_body,
        out_type=jax.ShapeDtypeStruct((NCORES, NPAD, M), jnp.float32),
        mesh=_mesh(),
        compiler_params=pltpu.CompilerParams(use_tc_tiling_on_sc=False),
        scratch_types=[
            pltpu.VMEM((CPT, CHUNK), jnp.int32),
            pltpu.VMEM((CPT, CHUNK), jnp.int32),
            pltpu.VMEM((2, CHUNK, M), jnp.float32),
            pltpu.VMEM((CHUNK, M), jnp.float32),
            pltpu.VMEM_SHARED((NPAD, M), jnp.float32),
            pltpu.SemaphoreType.DMA,
            pltpu.SemaphoreType.DMA,
        ],
    )(g, src2d, dst2d)


def _tc_b_body(degp_ref, x_ref, w1_ref, g1_ref, dinv_ref):
    d = degp_ref[0] + degp_ref[1] + 1.0
    dv = lax.rsqrt(d)
    h = jnp.dot(x_ref[...], w1_ref[...], preferred_element_type=jnp.float32)
    g1_ref[...] = dv * h
    dinv_ref[...] = dv


def _tc_b(degp, x, W1):
    return pl.pallas_call(
        _tc_b_body,
        grid=(N // RB,),
        in_specs=[
            pl.BlockSpec((NCORES, RB, 1), lambda i: (0, i, 0)),
            pl.BlockSpec((RB, D), lambda i: (i, 0)),
            pl.BlockSpec((D, M), lambda i: (0, 0)),
        ],
        out_specs=[
            pl.BlockSpec((RB, M), lambda i: (i, 0)),
            pl.BlockSpec((RB, 1), lambda i: (i, 0)),
        ],
        out_shape=[
            jax.ShapeDtypeStruct((N, M), jnp.float32),
            jax.ShapeDtypeStruct((N, 1), jnp.float32),
        ],
    )(degp, x, W1)


def _tc_d_body(sp_ref, g1_ref, dinv_ref, b1_ref, g2_ref):
    t = sp_ref[0] + sp_ref[1] + g1_ref[...]
    dv = dinv_ref[...]
    h = jnp.maximum(dv * t + b1_ref[...], 0.0)
    g2_ref[...] = dv * h


def _tc_d(sp, g1, dinv, b1row):
    return pl.pallas_call(
        _tc_d_body,
        grid=(N // RB,),
        in_specs=[
            pl.BlockSpec((NCORES, RB, M), lambda i: (0, i, 0)),
            pl.BlockSpec((RB, M), lambda i: (i, 0)),
            pl.BlockSpec((RB, 1), lambda i: (i, 0)),
            pl.BlockSpec((1, M), lambda i: (0, 0)),
        ],
        out_specs=pl.BlockSpec((RB, M), lambda i: (i, 0)),
        out_shape=jax.ShapeDtypeStruct((N, M), jnp.float32),
    )(sp, g1, dinv, b1row)


def _tc_f_body(sp_ref, g2_ref, dinv_ref, w2_ref, b2_ref, o_ref):
    t = dinv_ref[...] * (sp_ref[0] + sp_ref[1] + g2_ref[...])
    o_ref[...] = jnp.dot(t, w2_ref[...],
                         preferred_element_type=jnp.float32) + b2_ref[...]


def _tc_f(sp, g2, dinv, W2, b2row):
    return pl.pallas_call(
        _tc_f_body,
        grid=(N // RB,),
        in_specs=[
            pl.BlockSpec((NCORES, RB, M), lambda i: (0, i, 0)),
            pl.BlockSpec((RB, M), lambda i: (i, 0)),
            pl.BlockSpec((RB, 1), lambda i: (i, 0)),
            pl.BlockSpec((M, D), lambda i: (0, 0)),
            pl.BlockSpec((1, D), lambda i: (0, 0)),
        ],
        out_specs=pl.BlockSpec((RB, D), lambda i: (i, 0)),
        out_shape=jax.ShapeDtypeStruct((N, D), jnp.float32),
    )(sp, g2, dinv, W2, b2row)


def kernel(x, edge_index, W1, b1, W2, b2):
    src = edge_index[0]
    dst = edge_index[1]
    pad = EPAD - E
    src2d = jnp.concatenate(
        [src, jnp.zeros((pad,), jnp.int32)]).reshape(-1, CHUNK)
    dst2d = jnp.concatenate(
        [dst, jnp.full((pad,), N, jnp.int32)]).reshape(-1, CHUNK)

    degp = _sc_deg(dst2d)[:, :N].reshape(NCORES, N, 1)
    g1, dinv = _tc_b(degp, x, W1)
    s1 = _sc_agg(g1, src2d, dst2d)[:, :N, :]
    g2 = _tc_d(s1, g1, dinv, b1.reshape(1, M))
    s2 = _sc_agg(g2, src2d, dst2d)[:, :N, :]
    return _tc_f(s2, g2, dinv, W2, b2.reshape(1, D))
